# baseline (speedup 1.0000x reference)
import functools

import jax
import jax.numpy as jnp
from jax import lax
from jax.experimental import pallas as pl
from jax.experimental.pallas import tpu as pltpu
from jax.experimental.pallas import tpu_sc as plsc

N_ = 10000
K_ = 16
T_ = 16
E_ = N_ * K_
NODE_BLK = 400
EDGE_BLK = NODE_BLK * K_
ROW_BLK = 2000
EPS = 1e-5
F32 = jnp.float32


def _sc_gather(table, idx):
    info = plsc.get_sparse_core_info()
    nc, ns = info.num_cores, info.num_subcores
    nw = nc * ns
    e, d = idx.shape[0], table.shape[1]
    b_per_w = e // nw
    ch = 1000
    mesh = plsc.VectorSubcoreMesh(core_axis_name="c", subcore_axis_name="s")

    @functools.partial(
        pl.kernel, mesh=mesh,
        compiler_params=pltpu.CompilerParams(use_tc_tiling_on_sc=False),
        out_type=jax.ShapeDtypeStruct((e, d), F32),
        scratch_types=[
            pltpu.VMEM((ch,), jnp.int32),
            pltpu.VMEM((ch, d), F32),
            pltpu.SemaphoreType.DMA,
        ],
    )
    def gk(table_hbm, idx_hbm, out_hbm, idx_v, rows_v, sem):
        wid = lax.axis_index("s") * nc + lax.axis_index("c")
        base = wid * b_per_w
        for j in range(b_per_w // ch):
            off = base + j * ch
            pltpu.sync_copy(idx_hbm.at[pl.ds(off, ch)], idx_v)
            pltpu.async_copy(table_hbm.at[idx_v], rows_v, sem).wait()
            pltpu.sync_copy(rows_v, out_hbm.at[pl.ds(off, ch)])

    return gk(table, idx)


def _rb(x):
    return x.astype(jnp.bfloat16).astype(F32)


def _acc_stats(y, s_ref, q_ref):
    ps = jnp.sum(y, axis=0, keepdims=True)
    pq = jnp.sum(y * y, axis=0, keepdims=True)

    @pl.when(pl.program_id(0) == 0)
    def _():
        s_ref[...] = jnp.zeros_like(s_ref)
        q_ref[...] = jnp.zeros_like(q_ref)

    s_ref[...] += ps
    q_ref[...] += pq


def _full(shape):
    return pl.BlockSpec(shape, lambda i: tuple(0 for _ in shape))


def _lin_stats(x, w, b, blk):
    r, cin = x.shape
    cout = w.shape[1]

    def kfn(x_ref, w_ref, b_ref, y_ref, s_ref, q_ref):
        y = jnp.dot(_rb(x_ref[...]), w_ref[...], preferred_element_type=F32, precision=jax.lax.Precision.HIGHEST)
        y = y + b_ref[...]
        y_ref[...] = y
        _acc_stats(y, s_ref, q_ref)

    return pl.pallas_call(
        kfn,
        grid=(r // blk,),
        in_specs=[pl.BlockSpec((blk, cin), lambda i: (i, 0)),
                  _full((cin, cout)), _full((1, cout))],
        out_specs=[pl.BlockSpec((blk, cout), lambda i: (i, 0)),
                   _full((1, cout)), _full((1, cout))],
        out_shape=[jax.ShapeDtypeStruct((r, cout), F32),
                   jax.ShapeDtypeStruct((1, cout), F32),
                   jax.ShapeDtypeStruct((1, cout), F32)],
    )(x, w, b)


def _bn_relu(y, s, q, g, beta, count, blk, resid=None):
    r, c = y.shape
    inv = 1.0 / count

    def kfn(*refs):
        if resid is None:
            y_ref, s_ref, q_ref, g_ref, b_ref, o_ref = refs
        else:
            y_ref, s_ref, q_ref, g_ref, b_ref, r_ref, o_ref = refs
        m = s_ref[...] * inv
        v = q_ref[...] * inv - m * m
        sc = g_ref[...] / jnp.sqrt(v + EPS)
        out = jnp.maximum((y_ref[...] - m) * sc + b_ref[...], 0.0)
        if resid is not None:
            out = out + r_ref[...]
        o_ref[...] = out

    ins = [y, s, q, g, beta] + ([] if resid is None else [resid])
    in_specs = [pl.BlockSpec((blk, c), lambda i: (i, 0)),
                _full((1, c)), _full((1, c)), _full((1, c)), _full((1, c))]
    if resid is not None:
        in_specs.append(pl.BlockSpec((blk, c), lambda i: (i, 0)))
    return pl.pallas_call(
        kfn,
        grid=(r // blk,),
        in_specs=in_specs,
        out_specs=pl.BlockSpec((blk, c), lambda i: (i, 0)),
        out_shape=jax.ShapeDtypeStruct((r, c), F32),
    )(*ins)


def _etype_mp1_edge(gpts, pts_pad, pw, wc, wd, ww, b1, w2, b2,
                    wec, wen, wew, be):
    def kfn(gp_ref, pts_ref, pw_ref, wc_ref, wd_ref, ww_ref, b1_ref,
            w2_ref, b2_ref, wec_ref, wen_ref, wew_ref, be_ref,
            et_ref, ep_ref, s_ref, q_ref):
        ctr = jnp.broadcast_to(pts_ref[...][:, None, :], (NODE_BLK, K_, 16))
        ctr = ctr.reshape(EDGE_BLK, 16)
        gp = gp_ref[...]
        ctr_b = _rb(ctr)
        d_b = _rb(gp - ctr)
        gp_b = _rb(gp)
        pw_b = _rb(pw_ref[...])
        h = jnp.dot(ctr_b, wc_ref[...], preferred_element_type=F32, precision=jax.lax.Precision.HIGHEST)
        h = h + jnp.dot(d_b, wd_ref[...], preferred_element_type=F32, precision=jax.lax.Precision.HIGHEST)
        h = jnp.maximum(h + pw_b * ww_ref[...] + b1_ref[...], 0.0)
        logits = jnp.dot(_rb(h), w2_ref[...], preferred_element_type=F32, precision=jax.lax.Precision.HIGHEST)
        logits = logits + b2_ref[...]
        mx = jnp.max(logits, axis=1, keepdims=True)
        ex = jnp.exp(logits - mx)
        et_ref[...] = ex / jnp.sum(ex, axis=1, keepdims=True)
        ep = jnp.dot(ctr_b, wec_ref[...], preferred_element_type=F32, precision=jax.lax.Precision.HIGHEST)
        ep = ep + jnp.dot(gp_b, wen_ref[...], preferred_element_type=F32, precision=jax.lax.Precision.HIGHEST)
        ep = ep + pw_b * wew_ref[...] + be_ref[...]
        ep_ref[...] = ep
        _acc_stats(ep, s_ref, q_ref)

    return pl.pallas_call(
        kfn,
        grid=(E_ // EDGE_BLK,),
        in_specs=[pl.BlockSpec((EDGE_BLK, 16), lambda i: (i, 0)),
                  pl.BlockSpec((NODE_BLK, 16), lambda i: (i, 0)),
                  pl.BlockSpec((EDGE_BLK, 1), lambda i: (i, 0)),
                  _full((16, 64)), _full((16, 64)), _full((1, 64)),
                  _full((1, 64)), _full((64, T_)), _full((1, T_)),
                  _full((16, T_)), _full((16, T_)), _full((1, T_)),
                  _full((1, T_))],
        out_specs=[pl.BlockSpec((EDGE_BLK, T_), lambda i: (i, 0)),
                   pl.BlockSpec((EDGE_BLK, T_), lambda i: (i, 0)),
                   _full((1, T_)), _full((1, T_))],
        out_shape=[jax.ShapeDtypeStruct((E_, T_), F32),
                   jax.ShapeDtypeStruct((E_, T_), F32),
                   jax.ShapeDtypeStruct((1, T_), F32),
                   jax.ShapeDtypeStruct((1, T_), F32)],
    )(gpts, pts_pad, pw, wc, wd, ww, b1, w2, b2, wec, wen, wew, be)


def _mp_agg(et3, nbr3, selfx, wmsg, wself, bself):
    cin = nbr3.shape[2]
    cs = selfx.shape[1]

    def kfn(et_ref, nb_ref, sx_ref, wm_ref, ws_ref, bs_ref,
            y_ref, s_ref, q_ref):
        et = et_ref[...]
        nb = _rb(nb_ref[...])
        agg = jnp.zeros((NODE_BLK, 64), F32)
        for t in range(T_):
            s_t = jnp.sum(et[:, :, t:t + 1] * nb, axis=1)
            agg = agg + jnp.dot(s_t, wm_ref[t], preferred_element_type=F32, precision=jax.lax.Precision.HIGHEST)
        node = jnp.dot(_rb(sx_ref[...]), ws_ref[...], preferred_element_type=F32, precision=jax.lax.Precision.HIGHEST)
        node = node + bs_ref[...] + agg * (1.0 / K_)
        y_ref[...] = node
        _acc_stats(node, s_ref, q_ref)

    return pl.pallas_call(
        kfn,
        grid=(N_ // NODE_BLK,),
        in_specs=[pl.BlockSpec((NODE_BLK, K_, T_), lambda i: (i, 0, 0)),
                  pl.BlockSpec((NODE_BLK, K_, cin), lambda i: (i, 0, 0)),
                  pl.BlockSpec((NODE_BLK, cs), lambda i: (i, 0)),
                  _full((T_, cin, 64)), _full((cs, 64)), _full((1, 64))],
        out_specs=[pl.BlockSpec((NODE_BLK, 64), lambda i: (i, 0)),
                   _full((1, 64)), _full((1, 64))],
        out_shape=[jax.ShapeDtypeStruct((N_, 64), F32),
                   jax.ShapeDtypeStruct((1, 64), F32),
                   jax.ShapeDtypeStruct((1, 64), F32)],
    )(et3, nbr3, selfx, wmsg, wself, bself)


def _mp_edge(wf, nrows, nbr, wew, wec, wen, be):
    cn = nrows.shape[1]

    def kfn(wf_ref, nr_ref, nb_ref, wew_ref, wec_ref, wen_ref, be_ref,
            y_ref, s_ref, q_ref):
        ctr = jnp.broadcast_to(nr_ref[...][:, None, :], (NODE_BLK, K_, cn))
        ctr = ctr.reshape(EDGE_BLK, cn)
        ep = jnp.dot(_rb(wf_ref[...]), wew_ref[...], preferred_element_type=F32, precision=jax.lax.Precision.HIGHEST)
        ep = ep + jnp.dot(_rb(ctr), wec_ref[...], preferred_element_type=F32, precision=jax.lax.Precision.HIGHEST)
        ep = ep + jnp.dot(_rb(nb_ref[...]), wen_ref[...], preferred_element_type=F32, precision=jax.lax.Precision.HIGHEST)
        ep = ep + be_ref[...]
        y_ref[...] = ep
        _acc_stats(ep, s_ref, q_ref)

    return pl.pallas_call(
        kfn,
        grid=(E_ // EDGE_BLK,),
        in_specs=[pl.BlockSpec((EDGE_BLK, T_), lambda i: (i, 0)),
                  pl.BlockSpec((NODE_BLK, cn), lambda i: (i, 0)),
                  pl.BlockSpec((EDGE_BLK, cn), lambda i: (i, 0)),
                  _full((T_, T_)), _full((cn, T_)), _full((cn, T_)),
                  _full((1, T_))],
        out_specs=[pl.BlockSpec((EDGE_BLK, T_), lambda i: (i, 0)),
                   _full((1, T_)), _full((1, T_))],
        out_shape=[jax.ShapeDtypeStruct((E_, T_), F32),
                   jax.ShapeDtypeStruct((1, T_), F32),
                   jax.ShapeDtypeStruct((1, T_), F32)],
    )(wf, nrows, nbr, wew, wec, wen, be)


def _max_pool(x):
    r, c = x.shape

    def kfn(x_ref, o_ref):
        m = jnp.max(x_ref[...], axis=0, keepdims=True)

        @pl.when(pl.program_id(0) == 0)
        def _():
            o_ref[...] = m

        @pl.when(pl.program_id(0) != 0)
        def _():
            o_ref[...] = jnp.maximum(o_ref[...], m)

    return pl.pallas_call(
        kfn,
        grid=(r // ROW_BLK,),
        in_specs=[pl.BlockSpec((ROW_BLK, c), lambda i: (i, 0))],
        out_specs=_full((1, c)),
        out_shape=jax.ShapeDtypeStruct((1, c), F32),
    )(x)


def _ctx1(nf, gmax, wa, wb, b):
    def kfn(x_ref, g_ref, wa_ref, wb_ref, b_ref, y_ref, s_ref, q_ref):
        gterm = jnp.dot(_rb(g_ref[...]), wb_ref[...], preferred_element_type=F32, precision=jax.lax.Precision.HIGHEST)
        y = jnp.dot(_rb(x_ref[...]), wa_ref[...], preferred_element_type=F32, precision=jax.lax.Precision.HIGHEST)
        y = y + gterm + b_ref[...]
        y_ref[...] = y
        _acc_stats(y, s_ref, q_ref)

    return pl.pallas_call(
        kfn,
        grid=(N_ // ROW_BLK,),
        in_specs=[pl.BlockSpec((ROW_BLK, 64), lambda i: (i, 0)),
                  _full((1, 64)), _full((64, 64)), _full((64, 64)),
                  _full((1, 64))],
        out_specs=[pl.BlockSpec((ROW_BLK, 64), lambda i: (i, 0)),
                   _full((1, 64)), _full((1, 64))],
        out_shape=[jax.ShapeDtypeStruct((N_, 64), F32),
                   jax.ShapeDtypeStruct((1, 64), F32),
                   jax.ShapeDtypeStruct((1, 64), F32)],
    )(nf, gmax, wa, wb, b)


def _out_head(h, wo, bo):
    def kfn(h_ref, w_ref, b_ref, o_ref):
        o_ref[...] = jnp.dot(_rb(h_ref[...]), w_ref[...],
                             preferred_element_type=F32, precision=jax.lax.Precision.HIGHEST) + b_ref[...]

    return pl.pallas_call(
        kfn,
        grid=(N_ // ROW_BLK,),
        in_specs=[pl.BlockSpec((ROW_BLK, 64), lambda i: (i, 0)),
                  _full((64, 1)), _full((1, 1))],
        out_specs=pl.BlockSpec((ROW_BLK, 1), lambda i: (i, 0)),
        out_shape=jax.ShapeDtypeStruct((N_, 1), F32),
    )(h, wo, bo)


def _row(v):
    return v.reshape(1, -1).astype(F32)


def _pad_rows(m, rows):
    out = jnp.zeros((rows, m.shape[1]), F32)
    return out.at[: m.shape[0]].set(m)


def _mp_params(p, cin, cpad):
    win = p["edge"]["W"].shape[1] - 2 * cin
    wmsg = jnp.transpose(p["Wmsg"], (0, 2, 1)).astype(F32)
    if cpad != cin:
        wmsg = jnp.concatenate(
            [wmsg, jnp.zeros((T_, cpad - cin, 64), F32)], axis=1)
    we = p["edge"]["W"].astype(F32)
    return {
        "wself": _pad_rows(_rb(p["Wself"].T.astype(F32)), cpad),
        "bself": _row(p["bself"]),
        "wmsg": _rb(wmsg),
        "wew": _rb(we[:, :win].T),
        "wec": _pad_rows(_rb(we[:, win:win + cin].T), cpad),
        "wen": _pad_rows(_rb(we[:, win + cin:].T), cpad),
        "be": _row(p["edge"]["b"]),
        "gn": _row(p["bn_n"]["g"]), "bn": _row(p["bn_n"]["beta"]),
        "ge": _row(p["bn_e"]["g"]), "beb": _row(p["bn_e"]["beta"]),
    }


def _mp_block(nf, wf, et3, idx, mp, cin, nf_pad=None):
    table = nf if nf_pad is None else nf_pad
    nbr = _sc_gather(table, idx)
    cpad = table.shape[1]
    nbr3 = nbr.reshape(N_, K_, cpad)
    npre, ns, nq = _mp_agg(et3, nbr3, table, mp["wmsg"], mp["wself"],
                           mp["bself"])
    node = _bn_relu(npre, ns, nq, mp["gn"], mp["bn"], float(N_), ROW_BLK)
    epre, es, eq = _mp_edge(wf, table, nbr, mp["wew"], mp["wec"], mp["wen"],
                            mp["be"])
    edge = _bn_relu(epre, es, eq, mp["ge"], mp["beb"], float(E_), EDGE_BLK)
    return node, edge


def _cbr(x, p, blk, count, resid=None):
    y, s, q = _lin_stats(x, _rb(p["W"].T.astype(F32)), _row(p["b"]), blk)
    return _bn_relu(y, s, q, _row(p["g"]), _row(p["beta"]), count, blk,
                    resid=resid)


def _res_block(nf, wf, et3, idx, p):
    n1 = _cbr(nf, p["nconv1"], ROW_BLK, float(N_))
    w1 = _cbr(wf, p["wconv1"], EDGE_BLK, float(E_))
    mp = _mp_params(p["mp"], 64, 64)
    n2, w2 = _mp_block(n1, w1, et3, idx, mp, 64)
    nf_out = _cbr(n2, p["nconv2"], ROW_BLK, float(N_), resid=nf)
    wf_out = _cbr(w2, p["wconv2"], EDGE_BLK, float(E_), resid=wf)
    return nf_out, wf_out


def kernel(pts, pair_weight, nn_idx, params):
    ptsT = pts[0, :, :, 0].T.astype(F32)
    pts_pad = jnp.concatenate([ptsT, jnp.zeros((N_, 14), F32)], axis=1)
    idx = nn_idx[0].reshape(E_).astype(jnp.int32)
    pw = pair_weight[0, 0].reshape(E_, 1).astype(F32)

    w1 = params["etype"]["c1"]["W"].astype(F32)
    wc = _pad_rows(_rb(w1[:, :2].T), 16)
    wd = _pad_rows(_rb(w1[:, 2:4].T), 16)
    ww = _rb(_row(w1[:, 4]))
    b1 = _row(params["etype"]["c1"]["b"])
    w2 = _rb(params["etype"]["c2"]["W"].T.astype(F32))
    b2 = _row(params["etype"]["c2"]["b"])

    mp1 = _mp_params(params["mp1"], 2, 16)

    gpts = _sc_gather(pts_pad, idx)
    etype, epre, es, eq = _etype_mp1_edge(
        gpts, pts_pad, pw, wc, wd, ww, b1, w2, b2,
        mp1["wec"], mp1["wen"], mp1["wew"], mp1["be"])
    et3 = etype.reshape(N_, K_, T_)

    npre, ns, nq = _mp_agg(et3, gpts.reshape(N_, K_, 16), pts_pad,
                           mp1["wmsg"], mp1["wself"], mp1["bself"])
    nf = _bn_relu(npre, ns, nq, mp1["gn"], mp1["bn"], float(N_), ROW_BLK)
    wf = _bn_relu(epre, es, eq, mp1["ge"], mp1["beb"], float(E_), EDGE_BLK)

    nf, wf = _res_block(nf, wf, et3, idx, params["res1"])
    nf, wf = _res_block(nf, wf, et3, idx, params["res2"])

    gmax = _max_pool(nf)
    wctx = params["ctx1"]["W"].astype(F32)
    hpre, hs, hq = _ctx1(nf, gmax, _rb(wctx[:, :64].T), _rb(wctx[:, 64:].T),
                         _row(params["ctx1"]["b"]))
    h = _bn_relu(hpre, hs, hq, _row(params["ctx1"]["g"]),
                 _row(params["ctx1"]["beta"]), float(N_), ROW_BLK)
    out = _out_head(h, _rb(params["ctx2"]["W"].T.astype(F32)),
                    _row(params["ctx2"]["b"]))
    return out[:, 0][None, :]
